# baseline (speedup 1.0000x reference)
import jax
import jax.numpy as jnp
from jax import lax
from jax.experimental import pallas as pl
from jax.experimental.pallas import tpu as pltpu
from jax.experimental.pallas import tpu_sc as plsc

N, C, H, W = 4, 3, 544, 960
HW = H * W
NPTS = N * HW
NC, NS = 2, 16
PTS_W = NPTS // (NC * NS)
WPS = NS // 2
ROWS_W = PTS_W // W
PBLK = W
PNBLK = PTS_W // PBLK
BLK = 2 * W
NBLK = PTS_W // BLK


def _body(x_hbm, g_hbm, out_hbm, xp,
          sA0, sA1, sA2, xpbA, sB0, sB1, sB2, xpbB,
          gbA, i0A, i1A, wA0, wA1, wA2, wA3, vA0, vA1, outA,
          gbB, i0B, i1B, wB0, wB1, wB2, wB3, vB0, vB1, outB,
          semA, semB, gsA, gsB, osA, osB, psA, psB):
    cid = lax.axis_index("c")
    sid = lax.axis_index("s")
    n = 2 * cid + sid // WPS
    chunk = sid % WPS
    row_base = chunk * ROWS_W
    pix_base = n * HW + row_base * W

    pA = ((sA0, sA1, sA2), xpbA, semA, psA)
    pB = ((sB0, sB1, sB2), xpbB, semB, psB)

    def fire1(b, pbufs):
        strips, _, sem, _ = pbufs
        r0 = row_base + b
        for c in range(C):
            pltpu.async_copy(x_hbm.at[n, c, r0],
                             strips[c].at[pl.ds(0, W)], sem)
            tr = jnp.minimum(r0 + 1, H - 1)
            pltpu.async_copy(x_hbm.at[n, c, tr, pl.ds(0, 8)],
                             strips[c].at[pl.ds(W, 8)], sem)

    def xp_dst(b):
        return xp.at[pl.ds(pix_base + b * PBLK, PBLK)]

    def build1(b, pbufs):
        strips, xpb, sem, psem = pbufs
        @pl.when(b >= 2)
        def _():
            pltpu.make_async_copy(xpb, xp_dst(b - 2), psem).wait()

        r0 = row_base + b
        for c in range(C):
            pltpu.make_async_copy(x_hbm.at[n, c, r0],
                                  strips[c].at[pl.ds(0, W)], sem).wait()
            tr = jnp.minimum(r0 + 1, H - 1)
            pltpu.make_async_copy(x_hbm.at[n, c, tr, pl.ds(0, 8)],
                                  strips[c].at[pl.ds(W, 8)], sem).wait()

        def ilv(t, _):
            rows = lax.iota(jnp.int32, 16) + t * 16
            for c in range(C):
                v0 = strips[c][pl.ds(t * 16, 16)]
                v1 = strips[c][pl.ds(t * 16 + 1, 16)]
                plsc.store_scatter(xpb, [rows, jnp.full((16,), c, jnp.int32)], v0)
                plsc.store_scatter(
                    xpb, [rows, jnp.full((16,), c + C, jnp.int32)], v1)
            return _

        lax.fori_loop(0, PBLK // 16, ilv, None)
        pltpu.async_copy(xpb, xp_dst(b), psem)

    fire1(0, pA)

    def pipe1(g, _):
        b = 2 * g
        fire1(b + 1, pB)
        build1(b, pA)
        fire1(b + 2, pA)
        build1(b + 1, pB)
        return _

    lax.fori_loop(0, PNBLK // 2 - 1, pipe1, None)
    fire1(PNBLK - 1, pB)
    build1(PNBLK - 2, pA)
    build1(PNBLK - 1, pB)
    pltpu.make_async_copy(xpbA, xp_dst(PNBLK - 2), psA).wait()
    pltpu.make_async_copy(xpbB, xp_dst(PNBLK - 1), psB).wait()
    plsc.subcore_barrier()

    bufsA = (gbA, i0A, i1A, (wA0, wA1, wA2, wA3), vA0, vA1, outA, semA, gsA, osA)
    bufsB = (gbB, i0B, i1B, (wB0, wB1, wB2, wB3), vB0, vB1, outB, semB, gsB, osB)

    def grid_fetch(b, bufs):
        gbuf, _, _, _, _, _, _, _, gsem, _ = bufs
        r0 = row_base + 2 * b
        for rl in range(2):
            pltpu.async_copy(g_hbm.at[n, r0 + rl, 0],
                             gbuf.at[pl.ds(rl * W, W)], gsem)
            pltpu.async_copy(g_hbm.at[n, r0 + rl, 1],
                             gbuf.at[pl.ds(BLK + rl * W, W)], gsem)

    def fire(b, bufs):
        gbuf, i0v, i1v, wv, val0, val1, _, sem, gsem, _ = bufs
        r0 = row_base + 2 * b
        for rl in range(2):
            pltpu.make_async_copy(g_hbm.at[n, r0 + rl, 0],
                                  gbuf.at[pl.ds(rl * W, W)], gsem).wait()
            pltpu.make_async_copy(g_hbm.at[n, r0 + rl, 1],
                                  gbuf.at[pl.ds(BLK + rl * W, W)], gsem).wait()

        def idxw(j, _):
            sl = pl.ds(j * 16, 16)
            gx = gbuf[pl.ds(j * 16, 16)]
            gy = gbuf[pl.ds(BLK + j * 16, 16)]
            gx = jnp.minimum(jnp.maximum(gx, -1.0), 1.0)
            gy = jnp.minimum(jnp.maximum(gy, -1.0), 1.0)
            ix = (gx + 1.0) * 0.5 * (W - 1)
            iy = (gy + 1.0) * 0.5 * (H - 1)
            x0 = ix.astype(jnp.int32)
            y0 = iy.astype(jnp.int32)
            wx1 = ix - x0.astype(jnp.float32)
            wy1 = iy - y0.astype(jnp.float32)
            row = y0 * W + x0 + n * HW
            i0v[sl] = row
            i1v[sl] = row + jnp.where(y0 < H - 1, W, 0)
            wv[0][sl] = (1.0 - wy1) * (1.0 - wx1)
            wv[1][sl] = (1.0 - wy1) * wx1
            wv[2][sl] = wy1 * (1.0 - wx1)
            wv[3][sl] = wy1 * wx1
            return _

        lax.fori_loop(0, BLK // 16, idxw, None)
        pltpu.async_copy(xp.at[i0v], val0, sem)
        pltpu.async_copy(xp.at[i1v], val1, sem)

        @pl.when(b + 2 < NBLK)
        def _():
            grid_fetch(b + 2, bufs)

    def out_dst(b, comp, rl):
        return out_hbm.at[n, comp, row_base + 2 * b + rl]

    def drain(b, bufs):
        _, i0v, i1v, wv, val0, val1, outv, sem, _, osem = bufs
        @pl.when(b >= 2)
        def _():
            for comp in range(C):
                for rl in range(2):
                    pltpu.make_async_copy(
                        outv.at[pl.ds(comp * BLK + rl * W, W)],
                        out_dst(b - 2, comp, rl), osem).wait()

        pltpu.make_async_copy(xp.at[i0v], val0, sem).wait()
        pltpu.make_async_copy(xp.at[i1v], val1, sem).wait()

        def comb(j, _):
            sl = pl.ds(j * 16, 16)
            rows = lax.iota(jnp.int32, 16) + j * 16
            w00 = wv[0][sl]
            w01 = wv[1][sl]
            w10 = wv[2][sl]
            w11 = wv[3][sl]
            for comp in range(C):
                c0 = jnp.full((16,), comp, jnp.int32)
                c1 = jnp.full((16,), comp + C, jnp.int32)
                v00 = plsc.load_gather(val0, [rows, c0])
                v01 = plsc.load_gather(val0, [rows, c1])
                v10 = plsc.load_gather(val1, [rows, c0])
                v11 = plsc.load_gather(val1, [rows, c1])
                outv[pl.ds(comp * BLK + j * 16, 16)] = (
                    (v00 * w00 + v01 * w01) + (v10 * w10 + v11 * w11))
            return _

        lax.fori_loop(0, BLK // 16, comb, None)
        for comp in range(C):
            for rl in range(2):
                pltpu.async_copy(outv.at[pl.ds(comp * BLK + rl * W, W)],
                                 out_dst(b, comp, rl), osem)

    grid_fetch(0, bufsA)
    grid_fetch(1, bufsB)
    fire(0, bufsA)

    def pipe(g, _):
        b = 2 * g
        fire(b + 1, bufsB)
        drain(b, bufsA)
        fire(b + 2, bufsA)
        drain(b + 1, bufsB)
        return _

    lax.fori_loop(0, NBLK // 2 - 1, pipe, None)
    fire(NBLK - 1, bufsB)
    drain(NBLK - 2, bufsA)
    drain(NBLK - 1, bufsB)
    for comp in range(C):
        for rl in range(2):
            pltpu.make_async_copy(outA.at[pl.ds(comp * BLK + rl * W, W)],
                                  out_dst(NBLK - 2, comp, rl), osA).wait()
            pltpu.make_async_copy(outB.at[pl.ds(comp * BLK + rl * W, W)],
                                  out_dst(NBLK - 1, comp, rl), osB).wait()


def _p2_scratch():
    return [
        pltpu.VMEM((2 * BLK,), jnp.float32),
        pltpu.VMEM((BLK,), jnp.int32),
        pltpu.VMEM((BLK,), jnp.int32),
        pltpu.VMEM((BLK,), jnp.float32),
        pltpu.VMEM((BLK,), jnp.float32),
        pltpu.VMEM((BLK,), jnp.float32),
        pltpu.VMEM((BLK,), jnp.float32),
        pltpu.VMEM((BLK, 8), jnp.float32),
        pltpu.VMEM((BLK, 8), jnp.float32),
        pltpu.VMEM((C * BLK,), jnp.float32),
    ]


_sc_call = pl.kernel(
    _body,
    out_type=jax.ShapeDtypeStruct((N, C, H, W), jnp.float32),
    mesh=plsc.VectorSubcoreMesh(
        core_axis_name="c", subcore_axis_name="s",
        num_cores=NC, num_subcores=NS),
    scratch_types=[
        pltpu.HBM((NPTS, 8), jnp.float32),
        pltpu.VMEM((PBLK + 16,), jnp.float32),
        pltpu.VMEM((PBLK + 16,), jnp.float32),
        pltpu.VMEM((PBLK + 16,), jnp.float32),
        pltpu.VMEM((PBLK, 8), jnp.float32),
        pltpu.VMEM((PBLK + 16,), jnp.float32),
        pltpu.VMEM((PBLK + 16,), jnp.float32),
        pltpu.VMEM((PBLK + 16,), jnp.float32),
        pltpu.VMEM((PBLK, 8), jnp.float32),
    ] + _p2_scratch() + _p2_scratch() + [
        pltpu.SemaphoreType.DMA,
        pltpu.SemaphoreType.DMA,
        pltpu.SemaphoreType.DMA,
        pltpu.SemaphoreType.DMA,
        pltpu.SemaphoreType.DMA,
        pltpu.SemaphoreType.DMA,
        pltpu.SemaphoreType.DMA,
        pltpu.SemaphoreType.DMA,
    ],
    compiler_params=pltpu.CompilerParams(
        needs_layout_passes=False, use_tc_tiling_on_sc=False),
)


def kernel(x, grid):
    gt = jnp.transpose(grid, (0, 1, 3, 2))
    return _sc_call(x, gt)


# speedup vs baseline: 1.0312x; 1.0312x over previous
import jax
import jax.numpy as jnp
from jax import lax
from jax.experimental import pallas as pl
from jax.experimental.pallas import tpu as pltpu
from jax.experimental.pallas import tpu_sc as plsc

N, C, H, W = 4, 3, 544, 960
HW = H * W
NPTS = N * HW
NC, NS = 2, 16
PTS_W = NPTS // (NC * NS)
WPS = NS // 2
ROWS_W = PTS_W // W
PBLK = 2 * W
PNBLK = PTS_W // PBLK
BLK = W
NBLK = PTS_W // BLK
GCH = 120
NG = BLK // GCH


def _body(x_hbm, g_hbm, out_hbm, xp,
          sA0, sA1, sA2, xpbA, sB0, sB1, sB2, xpbB,
          gbA, i0A, i1A, wA0, wA1, wA2, wA3, vA0, vA1, outA,
          gbB, i0B, i1B, wB0, wB1, wB2, wB3, vB0, vB1, outB,
          semA, semB, gsA, gsB, osA, osB, psA, psB):
    cid = lax.axis_index("c")
    sid = lax.axis_index("s")
    n = 2 * cid + sid // WPS
    chunk = sid % WPS
    row_base = chunk * ROWS_W
    pix_base = n * HW + row_base * W

    pA = ((sA0, sA1, sA2), xpbA, semA, psA)
    pB = ((sB0, sB1, sB2), xpbB, semB, psB)

    def fire1(b, pbufs):
        strips, _, sem, _ = pbufs
        r0 = row_base + 2 * b
        for c in range(C):
            for r in range(2):
                pltpu.async_copy(x_hbm.at[n, c, r0 + r],
                                 strips[c].at[pl.ds(r * W, W)], sem)
            tr = jnp.minimum(r0 + 2, H - 1)
            pltpu.async_copy(x_hbm.at[n, c, tr, pl.ds(0, 8)],
                             strips[c].at[pl.ds(2 * W, 8)], sem)

    def xp_dst(b):
        return xp.at[pl.ds(pix_base + b * PBLK, PBLK)]

    def build1(b, pbufs):
        strips, xpb, sem, psem = pbufs
        @pl.when(b >= 2)
        def _():
            pltpu.make_async_copy(xpb, xp_dst(b - 2), psem).wait()

        r0 = row_base + 2 * b
        for c in range(C):
            for r in range(2):
                pltpu.make_async_copy(
                    x_hbm.at[n, c, r0 + r],
                    strips[c].at[pl.ds(r * W, W)], sem).wait()
            tr = jnp.minimum(r0 + 2, H - 1)
            pltpu.make_async_copy(x_hbm.at[n, c, tr, pl.ds(0, 8)],
                                  strips[c].at[pl.ds(2 * W, 8)], sem).wait()

        def ilv(t4, _):
          for u in range(4):
            t = t4 * 4 + u
            rows = lax.iota(jnp.int32, 16) + t * 16
            for c in range(C):
                v0 = strips[c][pl.ds(t * 16, 16)]
                v1 = strips[c][pl.ds(t * 16 + 1, 16)]
                plsc.store_scatter(xpb, [rows, jnp.full((16,), c, jnp.int32)], v0)
                plsc.store_scatter(
                    xpb, [rows, jnp.full((16,), c + C, jnp.int32)], v1)
          return _

        lax.fori_loop(0, PBLK // 64, ilv, None)
        pltpu.async_copy(xpb, xp_dst(b), psem)

    fire1(0, pA)

    def pipe1(g, _):
        b = 2 * g
        fire1(b + 1, pB)
        build1(b, pA)
        fire1(b + 2, pA)
        build1(b + 1, pB)
        return _

    lax.fori_loop(0, PNBLK // 2 - 1, pipe1, None)
    fire1(PNBLK - 1, pB)
    build1(PNBLK - 2, pA)
    build1(PNBLK - 1, pB)
    pltpu.make_async_copy(xpbA, xp_dst(PNBLK - 2), psA).wait()
    pltpu.make_async_copy(xpbB, xp_dst(PNBLK - 1), psB).wait()
    plsc.subcore_barrier()

    bufsA = (gbA, i0A, i1A, (wA0, wA1, wA2, wA3), vA0, vA1, outA, semA, gsA, osA)
    bufsB = (gbB, i0B, i1B, (wB0, wB1, wB2, wB3), vB0, vB1, outB, semB, gsB, osB)

    def grid_fetch(b, bufs):
        gbuf, _, _, _, _, _, _, _, gsem, _ = bufs
        r0 = row_base + b
        pltpu.async_copy(g_hbm.at[n, r0, 0], gbuf.at[pl.ds(0, W)], gsem)
        pltpu.async_copy(g_hbm.at[n, r0, 1], gbuf.at[pl.ds(W, W)], gsem)

    def fire(b, bufs):
        gbuf, i0v, i1v, wv, val0, val1, _, sem, gsem, _ = bufs
        r0 = row_base + b
        pltpu.make_async_copy(
            g_hbm.at[n, r0, 0], gbuf.at[pl.ds(0, W)], gsem).wait()
        pltpu.make_async_copy(
            g_hbm.at[n, r0, 1], gbuf.at[pl.ds(W, W)], gsem).wait()

        def idxw(j4, _):
          for u in range(4):
            j = j4 * 4 + u
            sl = pl.ds(j * 16, 16)
            gx = gbuf[pl.ds(j * 16, 16)]
            gy = gbuf[pl.ds(W + j * 16, 16)]
            gx = jnp.minimum(jnp.maximum(gx, -1.0), 1.0)
            gy = jnp.minimum(jnp.maximum(gy, -1.0), 1.0)
            ix = (gx + 1.0) * 0.5 * (W - 1)
            iy = (gy + 1.0) * 0.5 * (H - 1)
            x0 = ix.astype(jnp.int32)
            y0 = iy.astype(jnp.int32)
            wx1 = ix - x0.astype(jnp.float32)
            wy1 = iy - y0.astype(jnp.float32)
            row = y0 * W + x0 + n * HW
            i0v[sl] = row
            i1v[sl] = row + jnp.where(y0 < H - 1, W, 0)
            wv[0][sl] = (1.0 - wy1) * (1.0 - wx1)
            wv[1][sl] = (1.0 - wy1) * wx1
            wv[2][sl] = wy1 * (1.0 - wx1)
            wv[3][sl] = wy1 * wx1
          return _

        lax.fori_loop(0, BLK // 64, idxw, None)
        pltpu.async_copy(xp.at[i0v], val0, sem)
        pltpu.async_copy(xp.at[i1v], val1, sem)

        @pl.when(b + 2 < NBLK)
        def _():
            grid_fetch(b + 2, bufs)

    def out_dst(b, comp):
        return out_hbm.at[n, comp, row_base + b]

    def drain(b, bufs):
        _, i0v, i1v, wv, val0, val1, outv, sem, _, osem = bufs
        @pl.when(b >= 2)
        def _():
            for comp in range(C):
                pltpu.make_async_copy(
                    outv.at[pl.ds(comp * BLK, BLK)],
                    out_dst(b - 2, comp), osem).wait()

        pltpu.make_async_copy(xp.at[i0v], val0, sem).wait()
        pltpu.make_async_copy(xp.at[i1v], val1, sem).wait()

        def comb(j4, _):
          for u in range(4):
            j = j4 * 4 + u
            sl = pl.ds(j * 16, 16)
            rows = lax.iota(jnp.int32, 16) + j * 16
            w00 = wv[0][sl]
            w01 = wv[1][sl]
            w10 = wv[2][sl]
            w11 = wv[3][sl]
            for comp in range(C):
                c0 = jnp.full((16,), comp, jnp.int32)
                c1 = jnp.full((16,), comp + C, jnp.int32)
                v00 = plsc.load_gather(val0, [rows, c0])
                v01 = plsc.load_gather(val0, [rows, c1])
                v10 = plsc.load_gather(val1, [rows, c0])
                v11 = plsc.load_gather(val1, [rows, c1])
                outv[pl.ds(comp * BLK + j * 16, 16)] = (
                    (v00 * w00 + v01 * w01) + (v10 * w10 + v11 * w11))
          return _

        lax.fori_loop(0, BLK // 64, comb, None)
        for comp in range(C):
            pltpu.async_copy(outv.at[pl.ds(comp * BLK, BLK)],
                             out_dst(b, comp), osem)

    grid_fetch(0, bufsA)
    grid_fetch(1, bufsB)
    fire(0, bufsA)

    def pipe(g, _):
        b = 2 * g
        fire(b + 1, bufsB)
        drain(b, bufsA)
        fire(b + 2, bufsA)
        drain(b + 1, bufsB)
        return _

    lax.fori_loop(0, NBLK // 2 - 1, pipe, None)
    fire(NBLK - 1, bufsB)
    drain(NBLK - 2, bufsA)
    drain(NBLK - 1, bufsB)
    for comp in range(C):
        pltpu.make_async_copy(outA.at[pl.ds(comp * BLK, BLK)],
                              out_dst(NBLK - 2, comp), osA).wait()
        pltpu.make_async_copy(outB.at[pl.ds(comp * BLK, BLK)],
                              out_dst(NBLK - 1, comp), osB).wait()


def _p2_scratch():
    return [
        pltpu.VMEM((2 * W,), jnp.float32),
        pltpu.VMEM((BLK,), jnp.int32),
        pltpu.VMEM((BLK,), jnp.int32),
        pltpu.VMEM((BLK,), jnp.float32),
        pltpu.VMEM((BLK,), jnp.float32),
        pltpu.VMEM((BLK,), jnp.float32),
        pltpu.VMEM((BLK,), jnp.float32),
        pltpu.VMEM((BLK, 8), jnp.float32),
        pltpu.VMEM((BLK, 8), jnp.float32),
        pltpu.VMEM((C * BLK,), jnp.float32),
    ]


_sc_call = pl.kernel(
    _body,
    out_type=jax.ShapeDtypeStruct((N, C, H, W), jnp.float32),
    mesh=plsc.VectorSubcoreMesh(
        core_axis_name="c", subcore_axis_name="s",
        num_cores=NC, num_subcores=NS),
    scratch_types=[
        pltpu.HBM((NPTS, 8), jnp.float32),
        pltpu.VMEM((PBLK + 8,), jnp.float32),
        pltpu.VMEM((PBLK + 8,), jnp.float32),
        pltpu.VMEM((PBLK + 8,), jnp.float32),
        pltpu.VMEM((PBLK, 8), jnp.float32),
        pltpu.VMEM((PBLK + 8,), jnp.float32),
        pltpu.VMEM((PBLK + 8,), jnp.float32),
        pltpu.VMEM((PBLK + 8,), jnp.float32),
        pltpu.VMEM((PBLK, 8), jnp.float32),
    ] + _p2_scratch() + _p2_scratch() + [
        pltpu.SemaphoreType.DMA,
        pltpu.SemaphoreType.DMA,
        pltpu.SemaphoreType.DMA,
        pltpu.SemaphoreType.DMA,
        pltpu.SemaphoreType.DMA,
        pltpu.SemaphoreType.DMA,
        pltpu.SemaphoreType.DMA,
        pltpu.SemaphoreType.DMA,
    ],
    compiler_params=pltpu.CompilerParams(
        needs_layout_passes=False, use_tc_tiling_on_sc=False),
)


def kernel(x, grid):
    gt = jnp.transpose(grid, (0, 1, 3, 2))
    return _sc_call(x, gt)
